# baseline (speedup 1.0000x reference)
import functools

import jax
import jax.numpy as jnp
from jax import lax
from jax.experimental import pallas as pl
from jax.experimental.pallas import tpu as pltpu
from jax.experimental.pallas import tpu_sc as plsc

N_NODES = 10000
N_EDGES = 320000
D = 128
N_LAYERS = 5

NC = 2
NS = 16
K = 80
BLK = 16
NBLK = 8
CHUNKS = BLK * NBLK
E_PAD = NC * NS * CHUNKS * K
N_PAD = 10112
ROWS_PER_SUB = N_PAD // NS


def _sc_agg_body(h_hbm, src_hbm, dst_hbm, zeros_hbm, out_hbm,
                 isrc, idst, rows_v, acc, g0, g1, g2, g3,
                 s0, s1, s2, s3, i0, i1):
    c = lax.axis_index("c")
    s = lax.axis_index("s")
    gsems = (g0, g1, g2, g3)
    ssems = (s0, s1, s2, s3)
    isems = (i0, i1)

    pltpu.sync_copy(zeros_hbm.at[pl.ds(s * ROWS_PER_SUB, ROWS_PER_SUB)],
                    acc.at[pl.ds(s * ROWS_PER_SUB, ROWS_PER_SUB)])
    plsc.subcore_barrier()

    def idx_start(m):
        p = m % 2
        pltpu.async_copy(src_hbm.at[c, s, m], isrc.at[p], isems[p])
        pltpu.async_copy(dst_hbm.at[c, s, m], idst.at[p], isems[p])

    def idx_wait(m):
        p = m % 2
        pltpu.make_async_copy(src_hbm.at[c, s, 0], isrc.at[p],
                              isems[p]).wait()
        pltpu.make_async_copy(dst_hbm.at[c, s, 0], idst.at[p],
                              isems[p]).wait()

    def g_start(p, r, t):
        pltpu.async_copy(h_hbm.at[isrc.at[p, r]], rows_v.at[t], gsems[t])

    def g_wait(t):
        pltpu.make_async_copy(h_hbm.at[isrc.at[0, 0]], rows_v.at[t],
                              gsems[t]).wait()

    def s_start(p, r, t):
        pltpu.async_copy(rows_v.at[t], acc.at[idst.at[p, r]], ssems[t],
                         add=True)

    def s_wait(t):
        pltpu.make_async_copy(rows_v.at[t], acc.at[idst.at[0, 0]],
                              ssems[t]).wait()

    pltpu.sync_copy(src_hbm.at[c, s, 0], isrc.at[0])
    pltpu.sync_copy(dst_hbm.at[c, s, 0], idst.at[0])
    g_start(0, 0, 0)
    g_start(0, 1, 1)

    def make_step(ib, ib2):
        def one(r, t, r2, wait_scat=True, issue=True):
            if wait_scat:
                s_wait((t + 2) % 4)
            g_wait(t)
            s_start(ib, r, t)
            if issue:
                g_start(ib2, r2, (t + 2) % 4)
        return one

    for m in range(NBLK):
        ib = m % 2
        step0 = make_step(ib, ib)
        stepx = make_step(ib, 1 - ib)
        if m == 0:
            step0(0, 0, 2, wait_scat=False)
            step0(1, 1, 3, wait_scat=False)
        else:
            step0(0, 0, 2)
            step0(1, 1, 3)
        lo = 2
        if m < NBLK - 1:
            idx_start(m + 1)

        def fori_body(i, carry, _step=step0, _lo=lo):
            base = _lo + 4 * i
            for k in range(4):
                _step(base + k, (_lo + k) % 4, base + k + 2)
            return carry

        lax.fori_loop(0, 3, fori_body, 0)
        if m < NBLK - 1:
            idx_wait(m + 1)
            stepx(14, 2, 0)
            stepx(15, 3, 1)
        else:
            step0(14, 2, 0, issue=False)
            step0(15, 3, 1, issue=False)
    s_wait(2)
    s_wait(3)
    plsc.subcore_barrier()

    pltpu.sync_copy(acc.at[pl.ds(s * ROWS_PER_SUB, ROWS_PER_SUB)],
                    out_hbm.at[c, pl.ds(s * ROWS_PER_SUB, ROWS_PER_SUB)])


_sc_agg = functools.partial(
    pl.kernel,
    out_type=jax.ShapeDtypeStruct((NC, N_PAD, D), jnp.float32),
    mesh=plsc.VectorSubcoreMesh(core_axis_name="c", subcore_axis_name="s"),
    scratch_types=[
        pltpu.VMEM((2, BLK, K), jnp.int32),
        pltpu.VMEM((2, BLK, K), jnp.int32),
        pltpu.VMEM((4, K, D), jnp.float32),
        pltpu.VMEM_SHARED((N_PAD, D), jnp.float32),
    ] + [pltpu.SemaphoreType.DMA] * 10,
)(_sc_agg_body)


def _tc_layer_body(p_ref, h_ref, wr_ref, br_ref, wk_ref, g_ref, be_ref,
                   o_ref, *, residual):
    agg = p_ref[0, :N_NODES, :] + p_ref[1, :N_NODES, :]
    h_in = h_ref[...]
    h = lax.dot_general(agg, wr_ref[...], (((1,), (1,)), ((), ())),
                        preferred_element_type=jnp.float32)
    h = h + lax.dot_general(h_in, wk_ref[...], (((1,), (1,)), ((), ())),
                            preferred_element_type=jnp.float32)
    h = h + br_ref[...]
    mean = jnp.mean(h, axis=0, keepdims=True)
    d = h - mean
    var = jnp.mean(d * d, axis=0, keepdims=True)
    h = d * lax.rsqrt(var + 1e-5) * g_ref[...] + be_ref[...]
    h = jnp.maximum(h, 0.0)
    if residual:
        h = h + h_in
    o_ref[...] = h


def _tc_layer(parts, h_in, wr, br, wk, g, be, residual):
    body = functools.partial(_tc_layer_body, residual=residual)
    return pl.pallas_call(
        body,
        out_shape=jax.ShapeDtypeStruct((N_NODES, D), jnp.float32),
    )(parts, h_in, wr, br, wk, g, be)


def kernel(x, edge_index, W_rel, b_rel, W_root, gamma, beta):
    src = edge_index[0].astype(jnp.int32)
    dst = edge_index[1].astype(jnp.int32)
    pad = E_PAD - N_EDGES
    fill = jnp.arange(pad, dtype=jnp.int32)
    src = jnp.concatenate([src, fill % N_NODES])
    dst = jnp.concatenate([dst, N_NODES + fill % (N_PAD - N_NODES)])
    src_r = src.reshape(NC, NS, NBLK, BLK, K)
    dst_r = dst.reshape(NC, NS, NBLK, BLK, K)
    zeros = jnp.zeros((N_PAD, D), jnp.float32)

    h = x
    for i in range(N_LAYERS):
        parts = _sc_agg(h, src_r, dst_r, zeros)
        h = _tc_layer(parts, h, W_rel[i], b_rel[i].reshape(1, D),
                      W_root[i], gamma[i].reshape(1, D),
                      beta[i].reshape(1, D), residual=(i > 0))
    return h


# speedup vs baseline: 1.1399x; 1.1399x over previous
import functools

import jax
import jax.numpy as jnp
from jax import lax
from jax.experimental import pallas as pl
from jax.experimental.pallas import tpu as pltpu
from jax.experimental.pallas import tpu_sc as plsc

N_NODES = 10000
N_EDGES = 320000
D = 128
N_LAYERS = 5

NC = 2
NS = 16
K = 80
BLK = 16
NBLK = 8
CHUNKS = BLK * NBLK
E_PAD = NC * NS * CHUNKS * K
N_PAD = 10112
ROWS_PER_SUB = N_PAD // NS


def _sc_agg_body(h_hbm, src_hbm, dst_hbm, zeros_hbm, out_hbm,
                 isrc, idst, rows_v, acc, g0, g1, g2, g3,
                 s0, s1, s2, s3, i0, i1):
    c = lax.axis_index("c")
    s = lax.axis_index("s")
    gsems = (g0, g1, g2, g3)
    ssems = (s0, s1, s2, s3)
    isems = (i0, i1)

    pltpu.sync_copy(zeros_hbm.at[pl.ds(s * ROWS_PER_SUB, ROWS_PER_SUB)],
                    acc.at[pl.ds(s * ROWS_PER_SUB, ROWS_PER_SUB)])
    plsc.subcore_barrier()

    def idx_start(m):
        p = m % 2
        pltpu.async_copy(src_hbm.at[c, s, m], isrc.at[p], isems[p])
        pltpu.async_copy(dst_hbm.at[c, s, m], idst.at[p], isems[p])

    def idx_wait(m):
        p = m % 2
        pltpu.make_async_copy(src_hbm.at[c, s, 0], isrc.at[p],
                              isems[p]).wait()
        pltpu.make_async_copy(dst_hbm.at[c, s, 0], idst.at[p],
                              isems[p]).wait()

    KH = K // 2

    def g_start(p, r, t):
        pltpu.async_copy(h_hbm.at[isrc.at[p, r, pl.ds(0, KH)]],
                         rows_v.at[t, pl.ds(0, KH)], gsems[t])
        pltpu.async_copy(h_hbm.at[isrc.at[p, r, pl.ds(KH, KH)]],
                         rows_v.at[t, pl.ds(KH, KH)], ssems[t])

    def g_wait(t):
        pltpu.make_async_copy(h_hbm.at[isrc.at[0, 0, pl.ds(0, KH)]],
                              rows_v.at[t, pl.ds(0, KH)], gsems[t]).wait()
        pltpu.make_async_copy(h_hbm.at[isrc.at[0, 0, pl.ds(0, KH)]],
                              rows_v.at[t, pl.ds(KH, KH)], ssems[t]).wait()

    def scatter(p, r, t):
        pltpu.sync_copy(rows_v.at[t], acc.at[idst.at[p, r]], add=True)

    pltpu.sync_copy(src_hbm.at[c, s, 0], isrc.at[0])
    pltpu.sync_copy(dst_hbm.at[c, s, 0], idst.at[0])
    for j in range(3):
        g_start(0, j, j)

    def make_step(ib, ib3):
        def one(r, t, r3, issue=True):
            g_wait(t)
            scatter(ib, r, t)
            if issue:
                g_start(ib3, r3, (t + 3) % 4)
        return one

    for m in range(NBLK):
        ib = m % 2
        step0 = make_step(ib, ib)
        if m < NBLK - 1:
            idx_start(m + 1)
        step0(0, 0, 3)

        def fori_body(i, carry, _step=step0):
            base = 1 + 4 * i
            for k in range(4):
                _step(base + k, (1 + k) % 4, base + k + 3)
            return carry

        lax.fori_loop(0, 3, fori_body, 0)
        if m < NBLK - 1:
            idx_wait(m + 1)
            stepx = make_step(ib, 1 - ib)
            for r in (13, 14, 15):
                stepx(r, r % 4, r - 13)
        else:
            for r in (13, 14, 15):
                step0(r, r % 4, 0, issue=False)
    plsc.subcore_barrier()

    pltpu.sync_copy(acc.at[pl.ds(s * ROWS_PER_SUB, ROWS_PER_SUB)],
                    out_hbm.at[c, pl.ds(s * ROWS_PER_SUB, ROWS_PER_SUB)])


_sc_agg = functools.partial(
    pl.kernel,
    out_type=jax.ShapeDtypeStruct((NC, N_PAD, D), jnp.float32),
    mesh=plsc.VectorSubcoreMesh(core_axis_name="c", subcore_axis_name="s"),
    scratch_types=[
        pltpu.VMEM((2, BLK, K), jnp.int32),
        pltpu.VMEM((2, BLK, K), jnp.int32),
        pltpu.VMEM((4, K, D), jnp.float32),
        pltpu.VMEM_SHARED((N_PAD, D), jnp.float32),
    ] + [pltpu.SemaphoreType.DMA] * 10,
)(_sc_agg_body)


def _tc_layer_body(p_ref, h_ref, wr_ref, br_ref, wk_ref, g_ref, be_ref,
                   o_ref, *, residual):
    agg = p_ref[0, :N_NODES, :] + p_ref[1, :N_NODES, :]
    h_in = h_ref[...]
    h = lax.dot_general(agg, wr_ref[...], (((1,), (1,)), ((), ())),
                        preferred_element_type=jnp.float32)
    h = h + lax.dot_general(h_in, wk_ref[...], (((1,), (1,)), ((), ())),
                            preferred_element_type=jnp.float32)
    h = h + br_ref[...]
    mean = jnp.mean(h, axis=0, keepdims=True)
    d = h - mean
    var = jnp.mean(d * d, axis=0, keepdims=True)
    h = d * lax.rsqrt(var + 1e-5) * g_ref[...] + be_ref[...]
    h = jnp.maximum(h, 0.0)
    if residual:
        h = h + h_in
    o_ref[...] = h


def _tc_layer(parts, h_in, wr, br, wk, g, be, residual):
    body = functools.partial(_tc_layer_body, residual=residual)
    return pl.pallas_call(
        body,
        out_shape=jax.ShapeDtypeStruct((N_NODES, D), jnp.float32),
    )(parts, h_in, wr, br, wk, g, be)


def kernel(x, edge_index, W_rel, b_rel, W_root, gamma, beta):
    src = edge_index[0].astype(jnp.int32)
    dst = edge_index[1].astype(jnp.int32)
    pad = E_PAD - N_EDGES
    fill = jnp.arange(pad, dtype=jnp.int32)
    src = jnp.concatenate([src, fill % N_NODES])
    dst = jnp.concatenate([dst, N_NODES + fill % (N_PAD - N_NODES)])
    src_r = src.reshape(NC, NS, NBLK, BLK, K)
    dst_r = dst.reshape(NC, NS, NBLK, BLK, K)
    zeros = jnp.zeros((N_PAD, D), jnp.float32)

    h = x
    for i in range(N_LAYERS):
        parts = _sc_agg(h, src_r, dst_r, zeros)
        h = _tc_layer(parts, h, W_rel[i], b_rel[i].reshape(1, D),
                      W_root[i], gamma[i].reshape(1, D),
                      beta[i].reshape(1, D), residual=(i > 0))
    return h


# speedup vs baseline: 1.1531x; 1.0116x over previous
import functools

import jax
import jax.numpy as jnp
from jax import lax
from jax.experimental import pallas as pl
from jax.experimental.pallas import tpu as pltpu
from jax.experimental.pallas import tpu_sc as plsc

N_NODES = 10000
N_EDGES = 320000
D = 128
N_LAYERS = 5

NC = 2
NS = 16
K = 80
BLK = 16
NBLK = 8
CHUNKS = BLK * NBLK
E_PAD = NC * NS * CHUNKS * K
N_PAD = 10112
ROWS_PER_SUB = N_PAD // NS


def _sc_agg_body(h_hbm, src_hbm, dst_hbm, zeros_hbm, out_hbm,
                 isrc, idst, rows_v, acc, g0, g1, g2, g3, i0, i1):
    c = lax.axis_index("c")
    s = lax.axis_index("s")
    gsems = (g0, g1, g2, g3)
    isems = (i0, i1)

    pltpu.sync_copy(zeros_hbm.at[pl.ds(s * ROWS_PER_SUB, ROWS_PER_SUB)],
                    acc.at[pl.ds(s * ROWS_PER_SUB, ROWS_PER_SUB)])
    plsc.subcore_barrier()

    def idx_start(m):
        p = m % 2
        pltpu.async_copy(src_hbm.at[c, s, m], isrc.at[p], isems[p])
        pltpu.async_copy(dst_hbm.at[c, s, m], idst.at[p], isems[p])

    def idx_wait(m):
        p = m % 2
        pltpu.make_async_copy(src_hbm.at[c, s, 0], isrc.at[p],
                              isems[p]).wait()
        pltpu.make_async_copy(dst_hbm.at[c, s, 0], idst.at[p],
                              isems[p]).wait()

    def g_start(p, r, t):
        pltpu.async_copy(h_hbm.at[isrc.at[p, r]], rows_v.at[t], gsems[t])

    def g_wait(t):
        pltpu.make_async_copy(h_hbm.at[isrc.at[0, 0]], rows_v.at[t],
                              gsems[t]).wait()

    def scatter(p, r, t):
        pltpu.sync_copy(rows_v.at[t], acc.at[idst.at[p, r]], add=True)

    pltpu.sync_copy(src_hbm.at[c, s, 0], isrc.at[0])
    pltpu.sync_copy(dst_hbm.at[c, s, 0], idst.at[0])
    for j in range(3):
        g_start(0, j, j)

    def make_step(ib, ib3):
        def one(r, t, r3, issue=True):
            g_wait(t)
            scatter(ib, r, t)
            if issue:
                g_start(ib3, r3, (t + 3) % 4)
        return one

    for m in range(NBLK):
        ib = m % 2
        step0 = make_step(ib, ib)
        if m < NBLK - 1:
            idx_start(m + 1)
        step0(0, 0, 3)

        def fori_body(i, carry, _step=step0):
            base = 1 + 4 * i
            for k in range(4):
                _step(base + k, (1 + k) % 4, base + k + 3)
            return carry

        lax.fori_loop(0, 3, fori_body, 0)
        if m < NBLK - 1:
            idx_wait(m + 1)
            stepx = make_step(ib, 1 - ib)
            for r in (13, 14, 15):
                stepx(r, r % 4, r - 13)
        else:
            for r in (13, 14, 15):
                step0(r, r % 4, 0, issue=False)
    plsc.subcore_barrier()

    pltpu.sync_copy(acc.at[pl.ds(s * ROWS_PER_SUB, ROWS_PER_SUB)],
                    out_hbm.at[c, pl.ds(s * ROWS_PER_SUB, ROWS_PER_SUB)])


_sc_agg = functools.partial(
    pl.kernel,
    out_type=jax.ShapeDtypeStruct((NC, N_PAD, D), jnp.float32),
    mesh=plsc.VectorSubcoreMesh(core_axis_name="c", subcore_axis_name="s"),
    scratch_types=[
        pltpu.VMEM((2, BLK, K), jnp.int32),
        pltpu.VMEM((2, BLK, K), jnp.int32),
        pltpu.VMEM((4, K, D), jnp.float32),
        pltpu.VMEM_SHARED((N_PAD, D), jnp.float32),
    ] + [pltpu.SemaphoreType.DMA] * 6,
)(_sc_agg_body)


def _tc_layer_body(p_ref, h_ref, wr_ref, br_ref, wk_ref, g_ref, be_ref,
                   o_ref, *, residual):
    agg = p_ref[0, :N_NODES, :] + p_ref[1, :N_NODES, :]
    h_in = h_ref[...]
    h = lax.dot_general(agg, wr_ref[...], (((1,), (1,)), ((), ())),
                        preferred_element_type=jnp.float32)
    h = h + lax.dot_general(h_in, wk_ref[...], (((1,), (1,)), ((), ())),
                            preferred_element_type=jnp.float32)
    h = h + br_ref[...]
    mean = jnp.mean(h, axis=0, keepdims=True)
    d = h - mean
    var = jnp.mean(d * d, axis=0, keepdims=True)
    h = d * lax.rsqrt(var + 1e-5) * g_ref[...] + be_ref[...]
    h = jnp.maximum(h, 0.0)
    if residual:
        h = h + h_in
    o_ref[...] = h


def _tc_layer(parts, h_in, wr, br, wk, g, be, residual):
    body = functools.partial(_tc_layer_body, residual=residual)
    return pl.pallas_call(
        body,
        out_shape=jax.ShapeDtypeStruct((N_NODES, D), jnp.float32),
    )(parts, h_in, wr, br, wk, g, be)


def kernel(x, edge_index, W_rel, b_rel, W_root, gamma, beta):
    src = edge_index[0].astype(jnp.int32)
    dst = edge_index[1].astype(jnp.int32)
    pad = E_PAD - N_EDGES
    fill = jnp.arange(pad, dtype=jnp.int32)
    src = jnp.concatenate([src, fill % N_NODES])
    dst = jnp.concatenate([dst, N_NODES + fill % (N_PAD - N_NODES)])
    src_r = src.reshape(NC, NS, NBLK, BLK, K)
    dst_r = dst.reshape(NC, NS, NBLK, BLK, K)
    zeros = jnp.zeros((N_PAD, D), jnp.float32)

    h = x
    for i in range(N_LAYERS):
        parts = _sc_agg(h, src_r, dst_r, zeros)
        h = _tc_layer(parts, h, W_rel[i], b_rel[i].reshape(1, D),
                      W_root[i], gamma[i].reshape(1, D),
                      beta[i].reshape(1, D), residual=(i > 0))
    return h
